# baseline (speedup 1.0000x reference)
import functools

import jax
import jax.numpy as jnp
from jax import lax
from jax.experimental import pallas as pl
from jax.experimental.pallas import tpu as pltpu
from jax.experimental.pallas import tpu_sc as plsc

_B, _C, _NUM_BANDS = 32, 256, 64
_NC, _NS = 2, 16
_GATHER_WORKERS = 4
_ROWS_PER_W = _B // _GATHER_WORKERS


def _sc_gather_body(table_hbm, idx_hbm, gb_hbm, idx_v, rows_v, sem):
    wid = lax.axis_index("s") * _NC + lax.axis_index("c")

    @pl.when(wid == 0)
    def _():
        pltpu.sync_copy(idx_hbm, idx_v)
        pltpu.async_copy(table_hbm.at[idx_v], rows_v, sem).wait()
        pltpu.sync_copy(rows_v, gb_hbm)


@jax.jit
def _sc_gather(embed_weight, idx):
    mesh = plsc.VectorSubcoreMesh(core_axis_name="c", subcore_axis_name="s")
    return pl.kernel(
        _sc_gather_body,
        out_type=jax.ShapeDtypeStruct((_B, 2 * _C), jnp.float32),
        mesh=mesh,
        scratch_types=[
            pltpu.VMEM((_B,), jnp.int32),
            pltpu.VMEM((_B, 2 * _C), jnp.float32),
            pltpu.SemaphoreType.DMA,
        ],
    )(embed_weight, idx)


_NBUF = 4
_CHUNK = 2048


def _film_body(gb_ref, x_hbm, o_hbm, xb, ob, insems, outsems):
    M = x_hbm.shape[0]
    nchunk = M // _CHUNK
    rows_per_b = 64 * 64

    def in_copy(i, slot):
        return pltpu.make_async_copy(
            x_hbm.at[pl.ds(i * _CHUNK, _CHUNK)], xb.at[slot], insems.at[slot]
        )

    def out_copy(i, slot):
        return pltpu.make_async_copy(
            ob.at[slot], o_hbm.at[pl.ds(i * _CHUNK, _CHUNK)], outsems.at[slot]
        )

    for s in range(_NBUF):
        in_copy(s, s).start()

    def step(i, carry):
        slot = lax.rem(i, _NBUF)
        in_copy(i, slot).wait()

        @pl.when(i >= _NBUF)
        def _():
            out_copy(i - _NBUF, slot).wait()

        b = i // (rows_per_b // _CHUNK)
        g = 1.0 + gb_ref[pl.ds(b, 1), : _C]
        bt = gb_ref[pl.ds(b, 1), _C:]
        ob[slot] = xb[slot] * g + bt

        out_copy(i, slot).start()

        @pl.when(i + _NBUF < nchunk)
        def _():
            in_copy(i + _NBUF, slot).start()

        return carry

    lax.fori_loop(0, nchunk, step, 0)
    for k in range(_NBUF):
        i = nchunk - _NBUF + k
        out_copy(i, i % _NBUF).wait()


def _film(gb, x2d):
    M, C = x2d.shape
    return pl.pallas_call(
        _film_body,
        in_specs=[
            pl.BlockSpec(memory_space=pltpu.VMEM),
            pl.BlockSpec(memory_space=pl.ANY),
        ],
        out_specs=pl.BlockSpec(memory_space=pl.ANY),
        out_shape=jax.ShapeDtypeStruct(x2d.shape, x2d.dtype),
        scratch_shapes=[
            pltpu.VMEM((_NBUF, _CHUNK, C), jnp.float32),
            pltpu.VMEM((_NBUF, _CHUNK, C), jnp.float32),
            pltpu.SemaphoreType.DMA((_NBUF,)),
            pltpu.SemaphoreType.DMA((_NBUF,)),
        ],
    )(gb, x2d)


def kernel(x, band_idx, embed_weight):
    B, C, H, W = x.shape
    idx = band_idx.astype(jnp.int32)
    gb = _sc_gather(embed_weight, idx)
    x2d = jnp.transpose(x, (0, 2, 3, 1)).reshape(B * H * W, C)
    out2d = _film(gb, x2d)
    return jnp.transpose(out2d.reshape(B, H, W, C), (0, 3, 1, 2))


# speedup vs baseline: 1.0109x; 1.0109x over previous
import functools

import jax
import jax.numpy as jnp
from jax import lax
from jax.experimental import pallas as pl
from jax.experimental.pallas import tpu as pltpu
from jax.experimental.pallas import tpu_sc as plsc

_B, _C, _NUM_BANDS = 32, 256, 64
_NC, _NS = 2, 16
_GATHER_WORKERS = 4
_ROWS_PER_W = _B // _GATHER_WORKERS


def _sc_gather_body(table_hbm, idx_hbm, gb_hbm, idx_v, rows_v, sem):
    wid = lax.axis_index("s") * _NC + lax.axis_index("c")

    @pl.when(wid == 0)
    def _():
        pltpu.sync_copy(idx_hbm, idx_v)
        pltpu.async_copy(table_hbm.at[idx_v], rows_v, sem).wait()
        pltpu.sync_copy(rows_v, gb_hbm)


@jax.jit
def _sc_gather(embed_weight, idx):
    mesh = plsc.VectorSubcoreMesh(core_axis_name="c", subcore_axis_name="s", num_cores=1)
    return pl.kernel(
        _sc_gather_body,
        out_type=jax.ShapeDtypeStruct((_B, 2 * _C), jnp.float32),
        mesh=mesh,
        scratch_types=[
            pltpu.VMEM((_B,), jnp.int32),
            pltpu.VMEM((_B, 2 * _C), jnp.float32),
            pltpu.SemaphoreType.DMA,
        ],
    )(embed_weight, idx)


_NBUF = 4
_CHUNK = 2048


def _film_body(gb_ref, x_hbm, o_hbm, xb, ob, insems, outsems):
    M = x_hbm.shape[0]
    nchunk = M // _CHUNK
    rows_per_b = 64 * 64

    def in_copy(i, slot):
        return pltpu.make_async_copy(
            x_hbm.at[pl.ds(i * _CHUNK, _CHUNK)], xb.at[slot], insems.at[slot]
        )

    def out_copy(i, slot):
        return pltpu.make_async_copy(
            ob.at[slot], o_hbm.at[pl.ds(i * _CHUNK, _CHUNK)], outsems.at[slot]
        )

    for s in range(_NBUF):
        in_copy(s, s).start()

    def step(i, carry):
        slot = lax.rem(i, _NBUF)
        in_copy(i, slot).wait()

        @pl.when(i >= _NBUF)
        def _():
            out_copy(i - _NBUF, slot).wait()

        b = i // (rows_per_b // _CHUNK)
        g = 1.0 + gb_ref[pl.ds(b, 1), : _C]
        bt = gb_ref[pl.ds(b, 1), _C:]
        ob[slot] = xb[slot] * g + bt

        out_copy(i, slot).start()

        @pl.when(i + _NBUF < nchunk)
        def _():
            in_copy(i + _NBUF, slot).start()

        return carry

    lax.fori_loop(0, nchunk, step, 0)
    for k in range(_NBUF):
        i = nchunk - _NBUF + k
        out_copy(i, i % _NBUF).wait()


def _film(gb, x2d):
    M, C = x2d.shape
    return pl.pallas_call(
        _film_body,
        in_specs=[
            pl.BlockSpec(memory_space=pltpu.VMEM),
            pl.BlockSpec(memory_space=pl.ANY),
        ],
        out_specs=pl.BlockSpec(memory_space=pl.ANY),
        out_shape=jax.ShapeDtypeStruct(x2d.shape, x2d.dtype),
        scratch_shapes=[
            pltpu.VMEM((_NBUF, _CHUNK, C), jnp.float32),
            pltpu.VMEM((_NBUF, _CHUNK, C), jnp.float32),
            pltpu.SemaphoreType.DMA((_NBUF,)),
            pltpu.SemaphoreType.DMA((_NBUF,)),
        ],
    )(gb, x2d)


def kernel(x, band_idx, embed_weight):
    B, C, H, W = x.shape
    idx = band_idx.astype(jnp.int32)
    gb = _sc_gather(embed_weight, idx)
    x2d = jnp.transpose(x, (0, 2, 3, 1)).reshape(B * H * W, C)
    out2d = _film(gb, x2d)
    return jnp.transpose(out2d.reshape(B, H, W, C), (0, 3, 1, 2))


# speedup vs baseline: 1.0146x; 1.0036x over previous
import functools

import jax
import jax.numpy as jnp
from jax import lax
from jax.experimental import pallas as pl
from jax.experimental.pallas import tpu as pltpu
from jax.experimental.pallas import tpu_sc as plsc

_B, _C, _NUM_BANDS = 32, 256, 64
_NC, _NS = 2, 16
_GATHER_WORKERS = 4
_ROWS_PER_W = _B // _GATHER_WORKERS


def _sc_gather_body(table_hbm, idx_hbm, gb_hbm, idx_v, rows_v, sem):
    wid = lax.axis_index("s") * _NC + lax.axis_index("c")

    @pl.when(wid == 0)
    def _():
        pltpu.sync_copy(idx_hbm, idx_v)
        pltpu.async_copy(table_hbm.at[idx_v], rows_v, sem).wait()
        pltpu.sync_copy(rows_v, gb_hbm)


@jax.jit
def _sc_gather(embed_weight, idx):
    mesh = plsc.VectorSubcoreMesh(core_axis_name="c", subcore_axis_name="s", num_cores=1)
    return pl.kernel(
        _sc_gather_body,
        out_type=jax.ShapeDtypeStruct((_B, 2 * _C), jnp.float32),
        mesh=mesh,
        scratch_types=[
            pltpu.VMEM((_B,), jnp.int32),
            pltpu.VMEM((_B, 2 * _C), jnp.float32),
            pltpu.SemaphoreType.DMA,
        ],
    )(embed_weight, idx)


_NBUF = 4
_CHUNK = 4096


def _film_body(gb_ref, x_hbm, o_hbm, xb, ob, insems, outsems):
    M = x_hbm.shape[0]
    nchunk = M // _CHUNK
    rows_per_b = 64 * 64

    def in_copy(i, slot):
        return pltpu.make_async_copy(
            x_hbm.at[pl.ds(i * _CHUNK, _CHUNK)], xb.at[slot], insems.at[slot]
        )

    def out_copy(i, slot):
        return pltpu.make_async_copy(
            ob.at[slot], o_hbm.at[pl.ds(i * _CHUNK, _CHUNK)], outsems.at[slot]
        )

    for s in range(_NBUF):
        in_copy(s, s).start()

    def step(i, carry):
        slot = lax.rem(i, _NBUF)
        in_copy(i, slot).wait()

        @pl.when(i >= _NBUF)
        def _():
            out_copy(i - _NBUF, slot).wait()

        b = i // (rows_per_b // _CHUNK)
        g = 1.0 + gb_ref[pl.ds(b, 1), : _C]
        bt = gb_ref[pl.ds(b, 1), _C:]
        ob[slot] = xb[slot] * g + bt

        out_copy(i, slot).start()

        @pl.when(i + _NBUF < nchunk)
        def _():
            in_copy(i + _NBUF, slot).start()

        return carry

    lax.fori_loop(0, nchunk, step, 0)
    for k in range(_NBUF):
        i = nchunk - _NBUF + k
        out_copy(i, i % _NBUF).wait()


def _film(gb, x2d):
    M, C = x2d.shape
    return pl.pallas_call(
        _film_body,
        in_specs=[
            pl.BlockSpec(memory_space=pltpu.VMEM),
            pl.BlockSpec(memory_space=pl.ANY),
        ],
        out_specs=pl.BlockSpec(memory_space=pl.ANY),
        out_shape=jax.ShapeDtypeStruct(x2d.shape, x2d.dtype),
        scratch_shapes=[
            pltpu.VMEM((_NBUF, _CHUNK, C), jnp.float32),
            pltpu.VMEM((_NBUF, _CHUNK, C), jnp.float32),
            pltpu.SemaphoreType.DMA((_NBUF,)),
            pltpu.SemaphoreType.DMA((_NBUF,)),
        ],
    )(gb, x2d)


def kernel(x, band_idx, embed_weight):
    B, C, H, W = x.shape
    idx = band_idx.astype(jnp.int32)
    gb = _sc_gather(embed_weight, idx)
    x2d = jnp.transpose(x, (0, 2, 3, 1)).reshape(B * H * W, C)
    out2d = _film(gb, x2d)
    return jnp.transpose(out2d.reshape(B, H, W, C), (0, 3, 1, 2))
